# baseline (speedup 1.0000x reference)
import functools

import jax
import jax.numpy as jnp
from jax import lax
from jax.experimental import pallas as pl
from jax.experimental.pallas import tpu as pltpu
from jax.experimental.pallas import tpu_sc as plsc

K = 8192
D = 256
N = 8192
BETA = 0.25

TN = 256
TK = 1024
GRID_I = N // TN
GRID_J = K // TK
LOSS_SCALE = (1.0 + BETA) / (N * D)

CH = 128


def _vq_dist_body(lat_ref, emb_ref, l2_ref, e2_ref, inds_ref, loss_ref,
                  min_s, arg_s, acc_s):
    i = pl.program_id(0)
    j = pl.program_id(1)
    lat = lat_ref[...]
    emb = emb_ref[...]
    scores = lax.dot_general(
        lat, emb, (((1,), (1,)), ((), ())),
        preferred_element_type=jnp.float32)
    dist = (l2_ref[...] + e2_ref[...]) - 2.0 * scores
    m = jnp.min(dist, axis=1, keepdims=True)
    col = lax.broadcasted_iota(jnp.int32, (TN, TK), 1) + j * TK
    arg = jnp.min(jnp.where(dist == m, col, K), axis=1, keepdims=True)

    @pl.when(j == 0)
    def _():
        min_s[...] = m
        arg_s[...] = arg

    @pl.when(j > 0)
    def _():
        better = m < min_s[...]
        min_s[...] = jnp.where(better, m, min_s[...])
        arg_s[...] = jnp.where(better, arg, arg_s[...])

    @pl.when(j == GRID_J - 1)
    def _():
        inds_ref[...] = arg_s[...]
        tile_sum = jnp.sum(min_s[...])
        total = jnp.where(i == 0, 0.0, acc_s[0, 0]) + tile_sum
        acc_s[0, 0] = total

        @pl.when(i == GRID_I - 1)
        def _():
            loss_ref[0, 0] = total * LOSS_SCALE


def _tc_call_kwargs():
    return dict(
        grid=(GRID_I, GRID_J),
        in_specs=[
            pl.BlockSpec((TN, D), lambda i, j: (i, 0)),
            pl.BlockSpec((TK, D), lambda i, j: (j, 0)),
            pl.BlockSpec((TN, 1), lambda i, j: (i, 0)),
            pl.BlockSpec((1, TK), lambda i, j: (0, j)),
        ],
        out_specs=[
            pl.BlockSpec((TN, 1), lambda i, j: (i, 0)),
            pl.BlockSpec((1, 1), lambda i, j: (0, 0),
                         memory_space=pltpu.SMEM),
        ],
        out_shape=[
            jax.ShapeDtypeStruct((N, 1), jnp.int32),
            jax.ShapeDtypeStruct((1, 1), jnp.float32),
        ],
        scratch_shapes=[
            pltpu.VMEM((TN, 1), jnp.float32),
            pltpu.VMEM((TN, 1), jnp.int32),
            pltpu.SMEM((1, 1), jnp.float32),
        ],
    )


def _codebook_gather(embedding, idx3):
    info = plsc.get_sparse_core_info()
    nw = info.num_cores * info.num_subcores
    chunks = N // (nw * CH)
    mesh = plsc.VectorSubcoreMesh(core_axis_name="c", subcore_axis_name="s")

    @functools.partial(
        pl.kernel,
        out_type=jax.ShapeDtypeStruct((N, D), jnp.float32),
        mesh=mesh,
        scratch_types=[
            pltpu.VMEM((chunks, CH), jnp.int32),
            pltpu.VMEM((chunks, CH, D), jnp.float32),
            pltpu.SemaphoreType.DMA,
        ],
    )
    def gather_kernel(table_hbm, idx_hbm, out_hbm, idx_v, rows_v, sem):
        wid = lax.axis_index("s") * info.num_cores + lax.axis_index("c")
        base = wid * (chunks * CH)
        pltpu.sync_copy(idx_hbm.at[wid], idx_v)
        handles = [
            pltpu.async_copy(table_hbm.at[idx_v.at[c]], rows_v.at[c], sem)
            for c in range(chunks)
        ]
        for h in handles:
            h.wait()
        for c in range(chunks):
            pltpu.sync_copy(rows_v.at[c], out_hbm.at[pl.ds(base + c * CH, CH)])

    return gather_kernel(embedding, idx3)


def kernel(latents, embedding):
    flat = latents.reshape(N, D)
    l2 = jnp.sum(flat ** 2, axis=1, keepdims=True)
    e2 = jnp.sum(embedding ** 2, axis=1).reshape(1, K)
    inds, loss = pl.pallas_call(_vq_dist_body, **_tc_call_kwargs())(
        flat, embedding, l2, e2)
    info = plsc.get_sparse_core_info()
    nw = info.num_cores * info.num_subcores
    idx3 = inds.reshape(nw, N // (nw * CH), CH)
    quantized = _codebook_gather(embedding, idx3)
    return quantized.reshape(latents.shape), loss[0, 0]


# speedup vs baseline: 1.9177x; 1.9177x over previous
import functools

import jax
import jax.numpy as jnp
from jax import lax
from jax.experimental import pallas as pl
from jax.experimental.pallas import tpu as pltpu
from jax.experimental.pallas import tpu_sc as plsc

K = 8192
D = 256
N = 8192
BETA = 0.25

TN = 256
CK = 1024
CHUNKS = K // CK
GRID_I = N // TN
LOSS_SCALE = (1.0 + BETA) / (N * D)

CH = 128


def _vq_dist_body(lat_ref, emb_ref, l2_ref, e2_ref, inds_ref, loss_ref,
                  acc_s):
    i = pl.program_id(0)
    lat = lat_ref[...]
    l2 = l2_ref[...]
    m = None
    arg = None
    for c in range(CHUNKS):
        emb_c = emb_ref[pl.ds(c * CK, CK), :]
        scores = lax.dot_general(
            lat, emb_c, (((1,), (1,)), ((), ())),
            preferred_element_type=jnp.float32)
        e2_c = e2_ref[:, pl.ds(c * CK, CK)]
        dist = (l2 + e2_c) - 2.0 * scores
        m_c = jnp.min(dist, axis=1, keepdims=True)
        col = lax.broadcasted_iota(jnp.int32, (TN, CK), 1) + c * CK
        a_c = jnp.min(jnp.where(dist == m_c, col, K), axis=1, keepdims=True)
        if m is None:
            m, arg = m_c, a_c
        else:
            better = m_c < m
            arg = jnp.where(better, a_c, arg)
            m = jnp.where(better, m_c, m)

    inds_ref[...] = arg
    total = jnp.where(i == 0, 0.0, acc_s[0, 0]) + jnp.sum(m)
    acc_s[0, 0] = total

    @pl.when(i == GRID_I - 1)
    def _():
        loss_ref[0, 0] = total * LOSS_SCALE


def _tc_call_kwargs():
    return dict(
        grid=(GRID_I,),
        in_specs=[
            pl.BlockSpec((TN, D), lambda i: (i, 0)),
            pl.BlockSpec((K, D), lambda i: (0, 0)),
            pl.BlockSpec((TN, 1), lambda i: (i, 0)),
            pl.BlockSpec((1, K), lambda i: (0, 0)),
        ],
        out_specs=[
            pl.BlockSpec((TN, 1), lambda i: (i, 0)),
            pl.BlockSpec((1, 1), lambda i: (0, 0),
                         memory_space=pltpu.SMEM),
        ],
        out_shape=[
            jax.ShapeDtypeStruct((N, 1), jnp.int32),
            jax.ShapeDtypeStruct((1, 1), jnp.float32),
        ],
        scratch_shapes=[
            pltpu.SMEM((1, 1), jnp.float32),
        ],
    )


def _codebook_gather(embedding, idx3):
    info = plsc.get_sparse_core_info()
    nw = info.num_cores * info.num_subcores
    chunks = N // (nw * CH)
    mesh = plsc.VectorSubcoreMesh(core_axis_name="c", subcore_axis_name="s")

    @functools.partial(
        pl.kernel,
        out_type=jax.ShapeDtypeStruct((N, D), jnp.float32),
        mesh=mesh,
        scratch_types=[
            pltpu.VMEM((chunks, CH), jnp.int32),
            pltpu.VMEM((chunks, CH, D), jnp.float32),
            pltpu.SemaphoreType.DMA,
        ],
    )
    def gather_kernel(table_hbm, idx_hbm, out_hbm, idx_v, rows_v, sem):
        wid = lax.axis_index("s") * info.num_cores + lax.axis_index("c")
        base = wid * (chunks * CH)
        pltpu.sync_copy(idx_hbm.at[wid], idx_v)
        handles = [
            pltpu.async_copy(table_hbm.at[idx_v.at[c]], rows_v.at[c], sem)
            for c in range(chunks)
        ]
        for h in handles:
            h.wait()
        for c in range(chunks):
            pltpu.sync_copy(rows_v.at[c], out_hbm.at[pl.ds(base + c * CH, CH)])

    return gather_kernel(embedding, idx3)


def kernel(latents, embedding):
    flat = latents.reshape(N, D)
    l2 = jnp.sum(flat ** 2, axis=1, keepdims=True)
    e2 = jnp.sum(embedding ** 2, axis=1).reshape(1, K)
    inds, loss = pl.pallas_call(_vq_dist_body, **_tc_call_kwargs())(
        flat, embedding, l2, e2)
    info = plsc.get_sparse_core_info()
    nw = info.num_cores * info.num_subcores
    idx3 = inds.reshape(nw, N // (nw * CH), CH)
    quantized = _codebook_gather(embedding, idx3)
    return quantized.reshape(latents.shape), loss[0, 0]


# speedup vs baseline: 2.3560x; 1.2286x over previous
import functools

import jax
import jax.numpy as jnp
from jax import lax
from jax.experimental import pallas as pl
from jax.experimental.pallas import tpu as pltpu
from jax.experimental.pallas import tpu_sc as plsc

K = 8192
D = 256
N = 8192
BETA = 0.25

TN = 1024
CK = 1024
CHUNKS = K // CK
GRID_I = N // TN
LOSS_SCALE = (1.0 + BETA) / (N * D)

CH = 128


def _vq_dist_body(lat_ref, emb_ref, l2_ref, e2_ref, inds_ref, loss_ref,
                  acc_s):
    i = pl.program_id(0)
    lat = lat_ref[...]
    l2 = l2_ref[...]
    m = None
    arg = None
    for c in range(CHUNKS):
        emb_c = emb_ref[pl.ds(c * CK, CK), :]
        scores = lax.dot_general(
            emb_c, lat, (((1,), (1,)), ((), ())),
            preferred_element_type=jnp.float32)
        e2_c = e2_ref[pl.ds(c * CK, CK), :]
        dist = (l2 + e2_c) - 2.0 * scores
        m_c = jnp.min(dist, axis=0, keepdims=True)
        row = lax.broadcasted_iota(jnp.int32, (CK, TN), 0) + c * CK
        a_c = jnp.min(jnp.where(dist == m_c, row, K), axis=0, keepdims=True)
        if m is None:
            m, arg = m_c, a_c
        else:
            better = m_c < m
            arg = jnp.where(better, a_c, arg)
            m = jnp.where(better, m_c, m)

    inds_ref[...] = arg[None]
    total = jnp.where(i == 0, 0.0, acc_s[0, 0]) + jnp.sum(m)
    acc_s[0, 0] = total

    @pl.when(i == GRID_I - 1)
    def _():
        loss_ref[0, 0] = total * LOSS_SCALE


def _tc_call_kwargs():
    return dict(
        grid=(GRID_I,),
        in_specs=[
            pl.BlockSpec((TN, D), lambda i: (i, 0)),
            pl.BlockSpec((K, D), lambda i: (0, 0)),
            pl.BlockSpec((1, TN), lambda i: (0, i)),
            pl.BlockSpec((K, 1), lambda i: (0, 0)),
        ],
        out_specs=[
            pl.BlockSpec((1, 1, TN), lambda i: (i, 0, 0)),
            pl.BlockSpec((1, 1), lambda i: (0, 0),
                         memory_space=pltpu.SMEM),
        ],
        out_shape=[
            jax.ShapeDtypeStruct((GRID_I, 1, TN), jnp.int32),
            jax.ShapeDtypeStruct((1, 1), jnp.float32),
        ],
        scratch_shapes=[
            pltpu.SMEM((1, 1), jnp.float32),
        ],
    )


def _codebook_gather(embedding, idx3):
    info = plsc.get_sparse_core_info()
    nw = info.num_cores * info.num_subcores
    chunks = N // (nw * CH)
    mesh = plsc.VectorSubcoreMesh(core_axis_name="c", subcore_axis_name="s")

    @functools.partial(
        pl.kernel,
        out_type=jax.ShapeDtypeStruct((N, D), jnp.float32),
        mesh=mesh,
        scratch_types=[
            pltpu.VMEM((chunks, CH), jnp.int32),
            pltpu.VMEM((chunks, CH, D), jnp.float32),
            pltpu.SemaphoreType.DMA,
        ],
    )
    def gather_kernel(table_hbm, idx_hbm, out_hbm, idx_v, rows_v, sem):
        wid = lax.axis_index("s") * info.num_cores + lax.axis_index("c")
        base = wid * (chunks * CH)
        pltpu.sync_copy(idx_hbm.at[wid], idx_v)
        handles = [
            pltpu.async_copy(table_hbm.at[idx_v.at[c]], rows_v.at[c], sem)
            for c in range(chunks)
        ]
        for h in handles:
            h.wait()
        for c in range(chunks):
            pltpu.sync_copy(rows_v.at[c], out_hbm.at[pl.ds(base + c * CH, CH)])

    return gather_kernel(embedding, idx3)


def kernel(latents, embedding):
    flat = latents.reshape(N, D)
    l2 = jnp.sum(flat ** 2, axis=1).reshape(1, N)
    e2 = jnp.sum(embedding ** 2, axis=1).reshape(K, 1)
    inds, loss = pl.pallas_call(_vq_dist_body, **_tc_call_kwargs())(
        flat, embedding, l2, e2)
    info = plsc.get_sparse_core_info()
    nw = info.num_cores * info.num_subcores
    idx3 = inds.reshape(nw, N // (nw * CH), CH)
    quantized = _codebook_gather(embedding, idx3)
    return quantized.reshape(latents.shape), loss[0, 0]


# speedup vs baseline: 2.4733x; 1.0498x over previous
import functools

import jax
import jax.numpy as jnp
from jax import lax
from jax.experimental import pallas as pl
from jax.experimental.pallas import tpu as pltpu
from jax.experimental.pallas import tpu_sc as plsc

K = 8192
D = 256
N = 8192
BETA = 0.25

TN = 1024
CK = 1024
CHUNKS = K // CK
GRID_I = N // TN
LOSS_SCALE = (1.0 + BETA) / (N * D)

CH = 128


def _vq_dist_body(lat_ref, emb_ref, l2_ref, e2_ref, inds_ref, loss_ref,
                  acc_s):
    i = pl.program_id(0)
    lat = lat_ref[...]
    l2 = l2_ref[...]
    m = None
    arg = None
    row_f = lax.broadcasted_iota(jnp.int32, (CK, TN), 0).astype(jnp.float32)
    for c in range(CHUNKS):
        emb_c = emb_ref[pl.ds(c * CK, CK), :]
        scores = lax.dot_general(
            emb_c, lat, (((1,), (1,)), ((), ())),
            preferred_element_type=jnp.float32)
        e2_c = e2_ref[pl.ds(c * CK, CK), :]
        dist = (l2 + e2_c) - 2.0 * scores
        m_c = jnp.min(dist, axis=0, keepdims=True)
        a_c = jnp.min(jnp.where(dist == m_c, row_f, float(CK)),
                      axis=0, keepdims=True) + float(c * CK)
        if m is None:
            m, arg = m_c, a_c
        else:
            better = m_c < m
            arg = jnp.where(better, a_c, arg)
            m = jnp.where(better, m_c, m)

    inds_ref[...] = arg.astype(jnp.int32)[None]
    total = jnp.where(i == 0, 0.0, acc_s[0, 0]) + jnp.sum(m)
    acc_s[0, 0] = total

    @pl.when(i == GRID_I - 1)
    def _():
        loss_ref[0, 0] = total * LOSS_SCALE


def _tc_call_kwargs():
    return dict(
        grid=(GRID_I,),
        in_specs=[
            pl.BlockSpec((TN, D), lambda i: (i, 0)),
            pl.BlockSpec((K, D), lambda i: (0, 0)),
            pl.BlockSpec((1, TN), lambda i: (0, i)),
            pl.BlockSpec((K, 1), lambda i: (0, 0)),
        ],
        out_specs=[
            pl.BlockSpec((1, 1, TN), lambda i: (i, 0, 0)),
            pl.BlockSpec((1, 1), lambda i: (0, 0),
                         memory_space=pltpu.SMEM),
        ],
        out_shape=[
            jax.ShapeDtypeStruct((GRID_I, 1, TN), jnp.int32),
            jax.ShapeDtypeStruct((1, 1), jnp.float32),
        ],
        scratch_shapes=[
            pltpu.SMEM((1, 1), jnp.float32),
        ],
    )


def _codebook_gather(embedding, idx3):
    info = plsc.get_sparse_core_info()
    nw = info.num_cores * info.num_subcores
    chunks = N // (nw * CH)
    mesh = plsc.VectorSubcoreMesh(core_axis_name="c", subcore_axis_name="s")

    @functools.partial(
        pl.kernel,
        out_type=jax.ShapeDtypeStruct((N, D), jnp.float32),
        mesh=mesh,
        scratch_types=[
            pltpu.VMEM((chunks, CH), jnp.int32),
            pltpu.VMEM((chunks, CH, D), jnp.float32),
            pltpu.SemaphoreType.DMA,
        ],
    )
    def gather_kernel(table_hbm, idx_hbm, out_hbm, idx_v, rows_v, sem):
        wid = lax.axis_index("s") * info.num_cores + lax.axis_index("c")
        base = wid * (chunks * CH)
        pltpu.sync_copy(idx_hbm.at[wid], idx_v)
        handles = [
            pltpu.async_copy(table_hbm.at[idx_v.at[c]], rows_v.at[c], sem)
            for c in range(chunks)
        ]
        for h in handles:
            h.wait()
        for c in range(chunks):
            pltpu.sync_copy(rows_v.at[c], out_hbm.at[pl.ds(base + c * CH, CH)])

    return gather_kernel(embedding, idx3)


def kernel(latents, embedding):
    flat = latents.reshape(N, D)
    l2 = jnp.sum(flat ** 2, axis=1).reshape(1, N)
    e2 = jnp.sum(embedding ** 2, axis=1).reshape(K, 1)
    inds, loss = pl.pallas_call(_vq_dist_body, **_tc_call_kwargs())(
        flat, embedding, l2, e2)
    info = plsc.get_sparse_core_info()
    nw = info.num_cores * info.num_subcores
    idx3 = inds.reshape(nw, N // (nw * CH), CH)
    quantized = _codebook_gather(embedding, idx3)
    return quantized.reshape(latents.shape), loss[0, 0]


# speedup vs baseline: 2.7527x; 1.1130x over previous
import functools

import jax
import jax.numpy as jnp
from jax import lax
from jax.experimental import pallas as pl
from jax.experimental.pallas import tpu as pltpu
from jax.experimental.pallas import tpu_sc as plsc

K = 8192
D = 256
N = 8192
BETA = 0.25

TN = 1024
CK = 1024
CHUNKS = K // CK
GRID_I = N // TN
LOSS_SCALE = (1.0 + BETA) / (N * D)

CH = 128


def _vq_dist_body(lat_ref, emb_ref, l2_ref, e2_ref, inds_ref, loss_ref,
                  acc_s):
    i = pl.program_id(0)
    lat = lat_ref[...]
    l2 = l2_ref[...]
    m = None
    arg = None
    row_f = lax.broadcasted_iota(jnp.int32, (CK, TN), 0).astype(jnp.float32)
    for c in range(CHUNKS):
        emb_c = emb_ref[pl.ds(c * CK, CK), :]
        scores = lax.dot_general(
            emb_c, lat, (((1,), (1,)), ((), ())),
            preferred_element_type=jnp.float32)
        e2_c = e2_ref[pl.ds(c * CK, CK), :]
        dist = (l2 + e2_c) - 2.0 * scores
        m_c = jnp.min(dist, axis=0, keepdims=True)
        a_c = (jnp.argmin(dist, axis=0).astype(jnp.float32)[None]
               + float(c * CK))
        if m is None:
            m, arg = m_c, a_c
        else:
            better = m_c < m
            arg = jnp.where(better, a_c, arg)
            m = jnp.where(better, m_c, m)

    inds_ref[...] = arg.astype(jnp.int32)[None]
    total = jnp.where(i == 0, 0.0, acc_s[0, 0]) + jnp.sum(m)
    acc_s[0, 0] = total

    @pl.when(i == GRID_I - 1)
    def _():
        loss_ref[0, 0] = total * LOSS_SCALE


def _tc_call_kwargs():
    return dict(
        grid=(GRID_I,),
        in_specs=[
            pl.BlockSpec((TN, D), lambda i: (i, 0)),
            pl.BlockSpec((K, D), lambda i: (0, 0)),
            pl.BlockSpec((1, TN), lambda i: (0, i)),
            pl.BlockSpec((K, 1), lambda i: (0, 0)),
        ],
        out_specs=[
            pl.BlockSpec((1, 1, TN), lambda i: (i, 0, 0)),
            pl.BlockSpec((1, 1), lambda i: (0, 0),
                         memory_space=pltpu.SMEM),
        ],
        out_shape=[
            jax.ShapeDtypeStruct((GRID_I, 1, TN), jnp.int32),
            jax.ShapeDtypeStruct((1, 1), jnp.float32),
        ],
        scratch_shapes=[
            pltpu.SMEM((1, 1), jnp.float32),
        ],
    )


def _codebook_gather(embedding, idx3):
    info = plsc.get_sparse_core_info()
    nw = info.num_cores * info.num_subcores
    chunks = N // (nw * CH)
    mesh = plsc.VectorSubcoreMesh(core_axis_name="c", subcore_axis_name="s")

    @functools.partial(
        pl.kernel,
        out_type=jax.ShapeDtypeStruct((N, D), jnp.float32),
        mesh=mesh,
        scratch_types=[
            pltpu.VMEM((chunks, CH), jnp.int32),
            pltpu.VMEM((chunks, CH, D), jnp.float32),
            pltpu.SemaphoreType.DMA,
        ],
    )
    def gather_kernel(table_hbm, idx_hbm, out_hbm, idx_v, rows_v, sem):
        wid = lax.axis_index("s") * info.num_cores + lax.axis_index("c")
        base = wid * (chunks * CH)
        pltpu.sync_copy(idx_hbm.at[wid], idx_v)
        handles = [
            pltpu.async_copy(table_hbm.at[idx_v.at[c]], rows_v.at[c], sem)
            for c in range(chunks)
        ]
        for h in handles:
            h.wait()
        for c in range(chunks):
            pltpu.sync_copy(rows_v.at[c], out_hbm.at[pl.ds(base + c * CH, CH)])

    return gather_kernel(embedding, idx3)


def kernel(latents, embedding):
    flat = latents.reshape(N, D)
    l2 = jnp.sum(flat ** 2, axis=1).reshape(1, N)
    e2 = jnp.sum(embedding ** 2, axis=1).reshape(K, 1)
    inds, loss = pl.pallas_call(_vq_dist_body, **_tc_call_kwargs())(
        flat, embedding, l2, e2)
    info = plsc.get_sparse_core_info()
    nw = info.num_cores * info.num_subcores
    idx3 = inds.reshape(nw, N // (nw * CH), CH)
    quantized = _codebook_gather(embedding, idx3)
    return quantized.reshape(latents.shape), loss[0, 0]
